# Initial kernel scaffold; baseline (speedup 1.0000x reference)
import jax
import jax.numpy as jnp
from jax.experimental import pallas as pl


def kernel(idx, targets, token_embedding_table):
    raise NotImplementedError("write your pallas kernel here")


# speedup vs baseline: 1.2337x; 1.2337x over previous
import functools

import jax
import jax.numpy as jnp
from jax import lax
from jax.experimental import pallas as pl
from jax.experimental.pallas import tpu as pltpu
from jax.experimental.pallas import tpu_sc as plsc

VOCAB = 1000
N_TOK = 1024 * 200
NW = 32
PER_W = N_TOK // NW
CH = 32
NCH = PER_W // CH
NB = 2


def _logz_body(table_ref, out_ref):
    t = table_ref[...]
    m = jnp.max(t, axis=1, keepdims=True)
    s = jnp.sum(jnp.exp(t - m), axis=1, keepdims=True)
    out_ref[...] = m + jnp.log(s)


def _logz(table):
    return pl.pallas_call(
        _logz_body,
        out_shape=jax.ShapeDtypeStruct((VOCAB, 1), jnp.float32),
    )(table)


def _finish_body(part_ref, out_ref):
    out_ref[...] = jnp.sum(part_ref[...]).reshape(1, 1) * (1.0 / N_TOK)


def _finish(partials):
    return pl.pallas_call(
        _finish_body,
        out_shape=jax.ShapeDtypeStruct((1, 1), jnp.float32),
    )(partials)


def _sc_body(idx_hbm, tgt_hbm, table_hbm, logz_hbm, out_hbm, part_hbm,
             idx_v, tgt_v, logz_v, acc_v, rows0, rows1, gsem, ssem0, ssem1):
    rows = (rows0, rows1)
    ssem = (ssem0, ssem1)
    wid = lax.axis_index("s") * 2 + lax.axis_index("c")
    tbase = wid * PER_W

    pltpu.sync_copy(idx_hbm.at[pl.ds(tbase, PER_W)], idx_v)
    pltpu.sync_copy(tgt_hbm.at[pl.ds(tbase, PER_W)], tgt_v)
    pltpu.sync_copy(logz_hbm, logz_v)

    lane = lax.iota(jnp.int32, 16)

    def chunk(g, b, acc, first):
        loc = g * CH
        base = tbase + loc
        @pl.when(jnp.logical_not(first))
        def _():
            pltpu.make_async_copy(
                rows[b], out_hbm.at[pl.ds(base - NB * CH, CH)], ssem[b]
            ).wait()
        pltpu.async_copy(
            table_hbm.at[idx_v.at[pl.ds(loc, CH)]], rows[b], gsem
        ).wait()
        pltpu.async_copy(rows[b], out_hbm.at[pl.ds(base, CH)], ssem[b])
        for t in range(CH // 16):
            r16 = lane + (t * 16)
            t16 = tgt_v[pl.ds(loc + t * 16, 16)]
            i16 = idx_v[pl.ds(loc + t * 16, 16)]
            tl = plsc.load_gather(rows[b], [r16, t16])
            lz = plsc.load_gather(logz_v, [i16])
            acc = acc + (lz - tl)
        return acc

    def body(g0, acc):
        g = g0 * NB
        acc = chunk(g, 0, acc, g0 == 0)
        acc = chunk(g + 1, 1, acc, g0 == 0)
        return acc

    acc = lax.fori_loop(0, NCH // NB, body, jnp.zeros((16,), jnp.float32))
    for b in range(NB):
        pltpu.make_async_copy(
            rows[b],
            out_hbm.at[pl.ds(tbase + (NCH - NB + b) * CH, CH)],
            ssem[b],
        ).wait()
    acc_v[...] = acc
    pltpu.sync_copy(acc_v, part_hbm.at[wid])


@functools.partial(jax.jit, donate_argnums=())
def _sc_gather(idx_flat, tgt_flat, table, logz):
    mesh = plsc.VectorSubcoreMesh(core_axis_name="c", subcore_axis_name="s")
    f = functools.partial(
        pl.kernel,
        mesh=mesh,
        compiler_params=pltpu.CompilerParams(
            use_tc_tiling_on_sc=False, needs_layout_passes=False
        ),
        out_type=[
            jax.ShapeDtypeStruct((N_TOK, VOCAB), jnp.float32),
            jax.ShapeDtypeStruct((NW, 16), jnp.float32),
        ],
        scratch_types=[
            pltpu.VMEM((PER_W,), jnp.int32),
            pltpu.VMEM((PER_W,), jnp.int32),
            pltpu.VMEM((VOCAB,), jnp.float32),
            pltpu.VMEM((16,), jnp.float32),
            pltpu.VMEM((CH, VOCAB), jnp.float32),
            pltpu.VMEM((CH, VOCAB), jnp.float32),
            pltpu.SemaphoreType.DMA,
            pltpu.SemaphoreType.DMA,
            pltpu.SemaphoreType.DMA,
        ],
    )(_sc_body)
    return f(idx_flat, tgt_flat, table, logz)


def kernel(idx, targets, token_embedding_table):
    idx_flat = idx.reshape(-1).astype(jnp.int32)
    tgt_flat = targets.reshape(-1).astype(jnp.int32)
    table = token_embedding_table.astype(jnp.float32)
    logz = _logz(table).reshape(-1)
    logits, partials = _sc_gather(idx_flat, tgt_flat, table, logz)
    loss = _finish(partials)[0, 0]
    return (logits, loss)


# speedup vs baseline: 1.2461x; 1.0100x over previous
import functools

import jax
import jax.numpy as jnp
from jax import lax
from jax.experimental import pallas as pl
from jax.experimental.pallas import tpu as pltpu
from jax.experimental.pallas import tpu_sc as plsc

VOCAB = 1000
N_TOK = 1024 * 200
NW = 32
PER_W = N_TOK // NW
CH = 16
NCH = PER_W // CH
NB = 4
LA = 2


def _logz_body(table_ref, out_ref):
    t = table_ref[...]
    m = jnp.max(t, axis=1, keepdims=True)
    s = jnp.sum(jnp.exp(t - m), axis=1, keepdims=True)
    out_ref[...] = m + jnp.log(s)


def _logz(table):
    return pl.pallas_call(
        _logz_body,
        out_shape=jax.ShapeDtypeStruct((VOCAB, 1), jnp.float32),
    )(table)


def _finish_body(part_ref, out_ref):
    out_ref[...] = jnp.sum(part_ref[...]).reshape(1, 1) * (1.0 / N_TOK)


def _finish(partials):
    return pl.pallas_call(
        _finish_body,
        out_shape=jax.ShapeDtypeStruct((1, 1), jnp.float32),
    )(partials)


def _sc_body(idx_hbm, tgt_hbm, table_hbm, logz_hbm, out_hbm, part_hbm,
             idx_v, tgt_v, logz_v, acc_v,
             rows0, rows1, rows2, rows3,
             gsem0, gsem1, gsem2, gsem3, ssem0, ssem1, ssem2, ssem3):
    rows = (rows0, rows1, rows2, rows3)
    gsem = (gsem0, gsem1, gsem2, gsem3)
    ssem = (ssem0, ssem1, ssem2, ssem3)
    wid = lax.axis_index("s") * 2 + lax.axis_index("c")
    tbase = wid * PER_W

    pltpu.sync_copy(idx_hbm.at[pl.ds(tbase, PER_W)], idx_v)
    pltpu.sync_copy(tgt_hbm.at[pl.ds(tbase, PER_W)], tgt_v)
    pltpu.sync_copy(logz_hbm, logz_v)

    lane = lax.iota(jnp.int32, 16)

    def start_gather(g, b):
        pltpu.async_copy(
            table_hbm.at[idx_v.at[pl.ds(g * CH, CH)]], rows[b], gsem[b]
        )

    def wait_gather(g, b):
        pltpu.make_async_copy(
            table_hbm.at[idx_v.at[pl.ds(g * CH, CH)]], rows[b], gsem[b]
        ).wait()

    def start_scatter(g, b):
        pltpu.async_copy(rows[b], out_hbm.at[pl.ds(tbase + g * CH, CH)], ssem[b])

    def wait_scatter(g, b):
        pltpu.make_async_copy(
            rows[b], out_hbm.at[pl.ds(tbase + g * CH, CH)], ssem[b]
        ).wait()

    def loss_math(g, b, acc):
        loc = g * CH
        for t in range(CH // 16):
            r16 = lane + (t * 16)
            t16 = tgt_v[pl.ds(loc + t * 16, 16)]
            i16 = idx_v[pl.ds(loc + t * 16, 16)]
            tl = plsc.load_gather(rows[b], [r16, t16])
            lz = plsc.load_gather(logz_v, [i16])
            acc = acc + (lz - tl)
        return acc

    for c in range(LA):
        start_gather(c, c % NB)

    def body(g0, acc):
        for bb in range(NB):
            d = g0 * NB + bb
            bi = (bb + LA) % NB
            @pl.when(jnp.logical_and(d < NCH - LA, d >= NB - LA))
            def _():
                wait_scatter(d + LA - NB, bi)
            @pl.when(d < NCH - LA)
            def _():
                start_gather(d + LA, bi)
            wait_gather(d, bb)
            start_scatter(d, bb)
            acc = loss_math(d, bb, acc)
        return acc

    acc = lax.fori_loop(0, NCH // NB, body, jnp.zeros((16,), jnp.float32))
    for h in range(NCH - NB, NCH):
        wait_scatter(h, h % NB)
    acc_v[...] = acc
    pltpu.sync_copy(acc_v, part_hbm.at[wid])


@functools.partial(jax.jit, donate_argnums=())
def _sc_gather(idx_flat, tgt_flat, table, logz):
    mesh = plsc.VectorSubcoreMesh(core_axis_name="c", subcore_axis_name="s")
    f = functools.partial(
        pl.kernel,
        mesh=mesh,
        compiler_params=pltpu.CompilerParams(
            use_tc_tiling_on_sc=False, needs_layout_passes=False
        ),
        out_type=[
            jax.ShapeDtypeStruct((N_TOK, VOCAB), jnp.float32),
            jax.ShapeDtypeStruct((NW, 16), jnp.float32),
        ],
        scratch_types=[
            pltpu.VMEM((PER_W,), jnp.int32),
            pltpu.VMEM((PER_W,), jnp.int32),
            pltpu.VMEM((VOCAB,), jnp.float32),
            pltpu.VMEM((16,), jnp.float32),
            pltpu.VMEM((CH, VOCAB), jnp.float32),
            pltpu.VMEM((CH, VOCAB), jnp.float32),
            pltpu.VMEM((CH, VOCAB), jnp.float32),
            pltpu.VMEM((CH, VOCAB), jnp.float32),
            pltpu.SemaphoreType.DMA,
            pltpu.SemaphoreType.DMA,
            pltpu.SemaphoreType.DMA,
            pltpu.SemaphoreType.DMA,
            pltpu.SemaphoreType.DMA,
            pltpu.SemaphoreType.DMA,
            pltpu.SemaphoreType.DMA,
            pltpu.SemaphoreType.DMA,
        ],
    )(_sc_body)
    return f(idx_flat, tgt_flat, table, logz)


def kernel(idx, targets, token_embedding_table):
    idx_flat = idx.reshape(-1).astype(jnp.int32)
    tgt_flat = targets.reshape(-1).astype(jnp.int32)
    table = token_embedding_table.astype(jnp.float32)
    logz = _logz(table).reshape(-1)
    logits, partials = _sc_gather(idx_flat, tgt_flat, table, logz)
    loss = _finish(partials)[0, 0]
    return (logits, loss)


# speedup vs baseline: 1.6449x; 1.3201x over previous
import functools

import jax
import jax.numpy as jnp
from jax import lax
from jax.experimental import pallas as pl
from jax.experimental.pallas import tpu as pltpu
from jax.experimental.pallas import tpu_sc as plsc

VOCAB = 1000
CPAD = 1024
NPC = CPAD // 128
N_TOK = 1024 * 200
NW = 32
PER_W = N_TOK // NW
CH = 16
PPC = CH * NPC
NCH = PER_W // CH


def _logz_body(table_ref, out_ref):
    t = table_ref[...]
    m = jnp.max(t, axis=1, keepdims=True)
    s = jnp.sum(jnp.exp(t - m), axis=1, keepdims=True)
    out_ref[...] = m + jnp.log(s)


def _logz(table):
    return pl.pallas_call(
        _logz_body,
        out_shape=jax.ShapeDtypeStruct((VOCAB, 1), jnp.float32),
    )(table)


def _finish_body(part_ref, out_ref):
    out_ref[...] = jnp.sum(part_ref[...]).reshape(1, 1) * (1.0 / N_TOK)


def _finish(partials):
    return pl.pallas_call(
        _finish_body,
        out_shape=jax.ShapeDtypeStruct((1, 1), jnp.float32),
    )(partials)


def _sc_body(idx_hbm, tgt_hbm, tp_hbm, logz_hbm, out_hbm, part_hbm,
             idx_v, tgt_v, logz_v, acc_v,
             pidx0, pidx1, stag0, stag1, srow0, srow1,
             gsem0, gsem1, ssem0, ssem1):
    pidx = (pidx0, pidx1)
    stag = (stag0, stag1)
    srow = (srow0, srow1)
    gsem = (gsem0, gsem1)
    ssem = (ssem0, ssem1)
    wid = lax.axis_index("s") * 2 + lax.axis_index("c")
    tbase = wid * PER_W

    pltpu.sync_copy(idx_hbm.at[pl.ds(tbase, PER_W)], idx_v)
    pltpu.sync_copy(tgt_hbm.at[pl.ds(tbase, PER_W)], tgt_v)
    pltpu.sync_copy(logz_hbm, logz_v)

    lane = lax.iota(jnp.int32, 16)

    def compute_pidx(g, b):
        for j in range(PPC // 16):
            k = lane + (j * 16)
            rt = k // 64
            ct = (k // 8) % 8
            r = k % 8
            i16 = plsc.load_gather(idx_v, [g * CH + rt * 8 + r])
            pidx[b][pl.ds(j * 16, 16)] = i16 * NPC + ct

    def start_gather(g, b):
        pltpu.async_copy(tp_hbm.at[pidx[b]], stag[b], gsem[b])

    def wait_gather(g, b):
        pltpu.make_async_copy(tp_hbm.at[pidx[b]], stag[b], gsem[b]).wait()

    def start_scatter(g, b):
        pltpu.async_copy(srow[b], out_hbm.at[pl.ds(tbase + g * CH, CH)], ssem[b])

    def wait_scatter(g, b):
        pltpu.make_async_copy(
            srow[b], out_hbm.at[pl.ds(tbase + g * CH, CH)], ssem[b]
        ).wait()

    def retile(b):
        for r in range(CH):
            rt, rr = divmod(r, 8)
            for ct in range(NPC):
                p = rt * 64 + ct * 8 + rr
                nfull = 8 if ct < NPC - 1 else 6
                for cc in range(nfull):
                    srow[b][r, pl.ds(ct * 128 + cc * 16, 16)] = (
                        stag[b][p, pl.ds(cc * 16, 16)]
                    )
            p7 = rt * 64 + 7 * 8 + rr
            tail = stag[b][p7, pl.ds(96, 16)]
            plsc.store_scatter(
                srow[b],
                [jnp.full((16,), r, jnp.int32), 992 + lane],
                tail,
                mask=lane < 8,
            )

    def loss_math(g, b, acc):
        t16 = tgt_v[pl.ds(g * CH, 16)]
        i16 = idx_v[pl.ds(g * CH, 16)]
        rt = lane // 8
        r = lane % 8
        piece16 = rt * 64 + (t16 >> 7) * 8 + r
        col16 = t16 & 127
        tl = plsc.load_gather(stag[b], [piece16, col16])
        lz = plsc.load_gather(logz_v, [i16])
        return acc + (lz - tl)

    compute_pidx(0, 0)
    start_gather(0, 0)

    def body(g0, acc):
        for bb in range(2):
            d = g0 * 2 + bb
            b = bb
            b2 = (bb + 1) % 2
            @pl.when(d + 1 < NCH)
            def _():
                compute_pidx(d + 1, b2)
                start_gather(d + 1, b2)
            wait_gather(d, b)
            @pl.when(d >= 2)
            def _():
                wait_scatter(d - 2, b)
            retile(b)
            start_scatter(d, b)
            acc = loss_math(d, b, acc)
        return acc

    acc = lax.fori_loop(0, NCH // 2, body, jnp.zeros((16,), jnp.float32))
    for h in range(NCH - 2, NCH):
        wait_scatter(h, h % 2)
    acc_v[...] = acc
    pltpu.sync_copy(acc_v, part_hbm.at[wid])


@functools.partial(jax.jit, donate_argnums=())
def _sc_gather(idx_flat, tgt_flat, tpieces, logz):
    mesh = plsc.VectorSubcoreMesh(core_axis_name="c", subcore_axis_name="s")
    f = functools.partial(
        pl.kernel,
        mesh=mesh,
        compiler_params=pltpu.CompilerParams(
            use_tc_tiling_on_sc=True, needs_layout_passes=False
        ),
        out_type=[
            jax.ShapeDtypeStruct((N_TOK, VOCAB), jnp.float32),
            jax.ShapeDtypeStruct((NW, 16), jnp.float32),
        ],
        scratch_types=[
            pltpu.VMEM((PER_W,), jnp.int32),
            pltpu.VMEM((PER_W,), jnp.int32),
            pltpu.VMEM((VOCAB,), jnp.float32),
            pltpu.VMEM((16,), jnp.float32),
            pltpu.VMEM((PPC,), jnp.int32),
            pltpu.VMEM((PPC,), jnp.int32),
            pltpu.VMEM((PPC, 128), jnp.float32),
            pltpu.VMEM((PPC, 128), jnp.float32),
            pltpu.VMEM((CH, VOCAB), jnp.float32),
            pltpu.VMEM((CH, VOCAB), jnp.float32),
            pltpu.SemaphoreType.DMA,
            pltpu.SemaphoreType.DMA,
            pltpu.SemaphoreType.DMA,
            pltpu.SemaphoreType.DMA,
        ],
    )(_sc_body)
    return f(idx_flat, tgt_flat, tpieces, logz)


def kernel(idx, targets, token_embedding_table):
    idx_flat = idx.reshape(-1).astype(jnp.int32)
    tgt_flat = targets.reshape(-1).astype(jnp.int32)
    table = token_embedding_table.astype(jnp.float32)
    tpieces = jnp.pad(table, ((0, 0), (0, CPAD - VOCAB))).reshape(
        VOCAB * NPC, 128
    )
    logz = _logz(table).reshape(-1)
    logits, partials = _sc_gather(idx_flat, tgt_flat, tpieces, logz)
    loss = _finish(partials)[0, 0]
    return (logits, loss)


# speedup vs baseline: 1.6638x; 1.0115x over previous
import functools

import jax
import jax.numpy as jnp
from jax import lax
from jax.experimental import pallas as pl
from jax.experimental.pallas import tpu as pltpu
from jax.experimental.pallas import tpu_sc as plsc

VOCAB = 1000
CPAD = 1024
NPC = CPAD // 128
N_TOK = 1024 * 200
NW = 32
PER_W = N_TOK // NW
CH = 16
PPC = CH * NPC
NCH = PER_W // CH


def _logz_body(table_ref, out_ref):
    t = table_ref[...]
    m = jnp.max(t, axis=1, keepdims=True)
    s = jnp.sum(jnp.exp(t - m), axis=1, keepdims=True)
    out_ref[...] = m + jnp.log(s)


def _logz(table):
    return pl.pallas_call(
        _logz_body,
        out_shape=jax.ShapeDtypeStruct((VOCAB, 1), jnp.float32),
    )(table)


def _finish_body(part_ref, out_ref):
    out_ref[...] = jnp.sum(part_ref[...]).reshape(1, 1) * (1.0 / N_TOK)


def _finish(partials):
    return pl.pallas_call(
        _finish_body,
        out_shape=jax.ShapeDtypeStruct((1, 1), jnp.float32),
    )(partials)


def _sc_body(idx_hbm, tgt_hbm, tp_hbm, logz_hbm, out_hbm, part_hbm,
             idx_v, tgt_v, logz_v, acc_v,
             stag0, stag1, srow0, srow1,
             gsem0, gsem1, ssem0, ssem1):
    stag = (stag0, stag1)
    srow = (srow0, srow1)
    gsem = (gsem0, gsem1)
    ssem = (ssem0, ssem1)
    wid = lax.axis_index("s") * 2 + lax.axis_index("c")
    tbase = wid * PER_W

    pltpu.sync_copy(idx_hbm.at[pl.ds(tbase, PER_W)], idx_v)
    pltpu.sync_copy(tgt_hbm.at[pl.ds(tbase, PER_W)], tgt_v)
    pltpu.sync_copy(logz_hbm, logz_v)

    lane = lax.iota(jnp.int32, 16)

    def start_gather(g, b):
        pltpu.async_copy(
            tp_hbm.at[idx_v.at[pl.ds(g * CH, CH)]], stag[b], gsem[b]
        )

    def wait_gather(g, b):
        pltpu.make_async_copy(
            tp_hbm.at[idx_v.at[pl.ds(g * CH, CH)]], stag[b], gsem[b]
        ).wait()

    def start_scatter(g, b):
        pltpu.async_copy(srow[b], out_hbm.at[pl.ds(tbase + g * CH, CH)], ssem[b])

    def wait_scatter(g, b):
        pltpu.make_async_copy(
            srow[b], out_hbm.at[pl.ds(tbase + g * CH, CH)], ssem[b]
        ).wait()

    def retile(b):
        for r in range(CH):
            for ct in range(NPC):
                nfull = 8 if ct < NPC - 1 else 6
                for cc in range(nfull):
                    srow[b][r, pl.ds(ct * 128 + cc * 16, 16)] = (
                        stag[b][r, ct, pl.ds(cc * 16, 16)]
                    )
            tail = stag[b][r, NPC - 1, pl.ds(96, 16)]
            plsc.store_scatter(
                srow[b],
                [jnp.full((16,), r, jnp.int32), 992 + lane],
                tail,
                mask=lane < 8,
            )

    def loss_math(g, b, acc):
        t16 = tgt_v[pl.ds(g * CH, 16)]
        i16 = idx_v[pl.ds(g * CH, 16)]
        tl = plsc.load_gather(stag[b], [lane, t16 >> 7, t16 & 127])
        lz = plsc.load_gather(logz_v, [i16])
        return acc + (lz - tl)

    start_gather(0, 0)

    def body(g0, acc):
        for bb in range(2):
            d = g0 * 2 + bb
            b = bb
            b2 = (bb + 1) % 2
            @pl.when(d + 1 < NCH)
            def _():
                start_gather(d + 1, b2)
            wait_gather(d, b)
            @pl.when(d >= 2)
            def _():
                wait_scatter(d - 2, b)
            retile(b)
            start_scatter(d, b)
            acc = loss_math(d, b, acc)
        return acc

    acc = lax.fori_loop(0, NCH // 2, body, jnp.zeros((16,), jnp.float32))
    for h in range(NCH - 2, NCH):
        wait_scatter(h, h % 2)
    acc_v[...] = acc
    pltpu.sync_copy(acc_v, part_hbm.at[wid])


@functools.partial(jax.jit, donate_argnums=())
def _sc_gather(idx_flat, tgt_flat, tpieces, logz):
    mesh = plsc.VectorSubcoreMesh(core_axis_name="c", subcore_axis_name="s")
    f = functools.partial(
        pl.kernel,
        mesh=mesh,
        compiler_params=pltpu.CompilerParams(
            use_tc_tiling_on_sc=True, needs_layout_passes=False
        ),
        out_type=[
            jax.ShapeDtypeStruct((N_TOK, VOCAB), jnp.float32),
            jax.ShapeDtypeStruct((NW, 16), jnp.float32),
        ],
        scratch_types=[
            pltpu.VMEM((PER_W,), jnp.int32),
            pltpu.VMEM((PER_W,), jnp.int32),
            pltpu.VMEM((VOCAB,), jnp.float32),
            pltpu.VMEM((16,), jnp.float32),
            pltpu.VMEM((CH, NPC, 128), jnp.float32),
            pltpu.VMEM((CH, NPC, 128), jnp.float32),
            pltpu.VMEM((CH, VOCAB), jnp.float32),
            pltpu.VMEM((CH, VOCAB), jnp.float32),
            pltpu.SemaphoreType.DMA,
            pltpu.SemaphoreType.DMA,
            pltpu.SemaphoreType.DMA,
            pltpu.SemaphoreType.DMA,
        ],
    )(_sc_body)
    return f(idx_flat, tgt_flat, tpieces, logz)


def kernel(idx, targets, token_embedding_table):
    idx_flat = idx.reshape(-1).astype(jnp.int32)
    tgt_flat = targets.reshape(-1).astype(jnp.int32)
    table = token_embedding_table.astype(jnp.float32)
    tpieces = jnp.pad(table, ((0, 0), (0, CPAD - VOCAB))).reshape(
        VOCAB, NPC, 128
    )
    logz = _logz(table).reshape(-1)
    logits, partials = _sc_gather(idx_flat, tgt_flat, tpieces, logz)
    loss = _finish(partials)[0, 0]
    return (logits, loss)
